# baseline (speedup 1.0000x reference)
import functools

import jax
import jax.numpy as jnp
from jax import lax
from jax.experimental import pallas as pl
from jax.experimental.pallas import tpu as pltpu
from jax.experimental.pallas import tpu_sc as plsc

N_NODES_C = 10000
N_EDGES_C = 320000
CH = 128

_NC = 2
_NS = 16
_NW = _NC * _NS
_B = 128
_NCHUNK = 79
_EPT = _NCHUNK * _B
_NPAD = _NW * _EPT - N_EDGES_C
_NACC = N_NODES_C + 8


def _sc_body(x_hbm, row_hbm, col_hbm, zacc_hbm, zdeg_hbm,
             acc_out, deg_out,
             acc_sh, rbs, cbs, vs, degp, sems, semrs):
    c = lax.axis_index("c")
    s = lax.axis_index("s")
    wid = c * _NS + s
    base = pl.multiple_of(wid * _EPT, 8)

    pltpu.sync_copy(zdeg_hbm, degp)

    @pl.when(s == 0)
    def _():
        pltpu.sync_copy(zacc_hbm, acc_sh)

    plsc.subcore_barrier()

    ones16 = jnp.ones((16,), jnp.float32)

    def load_idx(j, b):
        off = pl.multiple_of(base + j * _B, 8)
        pltpu.async_copy(row_hbm.at[pl.ds(off, _B)], rbs[b], semrs[b])
        pltpu.async_copy(col_hbm.at[pl.ds(off, _B)], cbs[b], semrs[b])

    def wait_idx(j, b):
        off = pl.multiple_of(base + j * _B, 8)
        pltpu.make_async_copy(row_hbm.at[pl.ds(off, _B)], rbs[b], semrs[b]).wait()
        pltpu.make_async_copy(col_hbm.at[pl.ds(off, _B)], cbs[b], semrs[b]).wait()

    def gather(b, p):
        pltpu.async_copy(x_hbm.at[cbs[b]], vs[p], sems[p])

    def flush(b, p):
        rb = rbs[b]
        for i in range(_B // 16):
            plsc.addupdate_scatter(degp, [rb[pl.ds(i * 16, 16)]], ones16)
        pltpu.make_async_copy(x_hbm.at[cbs[b]], vs[p], sems[p]).wait()
        pltpu.sync_copy(vs[p], acc_sh.at[rb], add=True)

    for b in range(4):
        load_idx(b, b)
    wait_idx(0, 0)
    gather(0, 0)

    def step(k, carry):
        j = 4 * k
        for i in range(4):
            wait_idx(j + i + 1, (i + 1) % 4)
            gather((i + 1) % 4, (i + 1) % 2)
            flush(i % 4, i % 2)
            load_idx(j + i + 4, i % 4)
        return carry

    lax.fori_loop(0, (_NCHUNK - 3) // 4, step, 0)
    for i in range(3):
        if i < 2:
            wait_idx(76 + i + 1, i + 1)
            gather(i + 1, (i + 1) % 2)
        flush(i, i % 2)
    wait_idx(_NCHUNK, 3)

    d0 = pl.multiple_of(wid * _NACC, 8)
    pltpu.sync_copy(degp, deg_out.at[pl.ds(d0, _NACC)])

    plsc.subcore_barrier()

    @pl.when(s < 10)
    def _():
        for k in range(7):
            r0 = pl.multiple_of(s * 1000 + k * _B, 8)
            pltpu.sync_copy(acc_sh.at[pl.ds(r0, _B)], vs[0])
            pltpu.sync_copy(vs[0], acc_out.at[c, pl.ds(r0, _B)])
        r0 = pl.multiple_of(s * 1000 + 7 * _B, 8)
        pltpu.sync_copy(acc_sh.at[pl.ds(r0, 104)], vs[0].at[pl.ds(0, 104)])
        pltpu.sync_copy(vs[0].at[pl.ds(0, 104)], acc_out.at[c, pl.ds(r0, 104)])


@jax.jit
def _sc_scatter(x, row1, col1):
    zacc = jnp.zeros((_NACC, CH), jnp.float32)
    zdeg = jnp.zeros((_NACC,), jnp.float32)
    mesh = plsc.VectorSubcoreMesh(core_axis_name="c", subcore_axis_name="s")
    f = pl.kernel(
        _sc_body,
        out_type=[
            jax.ShapeDtypeStruct((_NC, N_NODES_C, CH), jnp.float32),
            jax.ShapeDtypeStruct((_NW * _NACC,), jnp.float32),
        ],
        mesh=mesh,
        compiler_params=pltpu.CompilerParams(needs_layout_passes=False),
        scratch_types=[
            pltpu.VMEM_SHARED((_NACC, CH), jnp.float32),
            [pltpu.VMEM((_B,), jnp.int32) for _ in range(4)],
            [pltpu.VMEM((_B,), jnp.int32) for _ in range(4)],
            [pltpu.VMEM((_B, CH), jnp.float32) for _ in range(2)],
            pltpu.VMEM((_NACC,), jnp.float32),
            [pltpu.SemaphoreType.DMA for _ in range(2)],
            [pltpu.SemaphoreType.DMA for _ in range(4)],
        ],
    )
    return f(x, row1, col1, zacc, zdeg)


def _tc_body(acc_ref, deg_ref, x_ref, w_ref, root_ref, bias_ref, o_ref):
    deg = jnp.maximum(jnp.sum(deg_ref[...], axis=1, keepdims=True), 1.0)
    agg = (acc_ref[0] + acc_ref[1]) / deg
    o_ref[...] = (
        jnp.dot(agg, w_ref[...], preferred_element_type=jnp.float32)
        + jnp.dot(x_ref[...], root_ref[...], preferred_element_type=jnp.float32)
        + bias_ref[...]
    )


@jax.jit
def _tc_combine(acc2, degt, x, weight, root, bias2):
    R = 1000
    grid = (N_NODES_C // R,)
    return pl.pallas_call(
        _tc_body,
        grid=grid,
        in_specs=[
            pl.BlockSpec((_NC, R, CH), lambda i: (0, i, 0)),
            pl.BlockSpec((R, _NW), lambda i: (i, 0)),
            pl.BlockSpec((R, CH), lambda i: (i, 0)),
            pl.BlockSpec((CH, CH), lambda i: (0, 0)),
            pl.BlockSpec((CH, CH), lambda i: (0, 0)),
            pl.BlockSpec((1, CH), lambda i: (0, 0)),
        ],
        out_specs=pl.BlockSpec((R, CH), lambda i: (i, 0)),
        out_shape=jax.ShapeDtypeStruct((N_NODES_C, CH), jnp.float32),
    )(acc2, degt, x, weight, root, bias2)


def kernel(x, edge_index, weight, root, bias):
    row = edge_index[0].astype(jnp.int32)
    col = edge_index[1].astype(jnp.int32)
    pad_row = N_NODES_C + (jnp.arange(_NPAD + _B, dtype=jnp.int32) % 8)
    row1 = jnp.concatenate([row, pad_row])
    col1 = jnp.concatenate([col, jnp.zeros((_NPAD + _B,), jnp.int32)])
    acc2, degf = _sc_scatter(x, row1, col1)
    degt = degf.reshape(_NW, _NACC).T
    return _tc_combine(acc2, degt, x, weight, root, bias[None, :])


# speedup vs baseline: 1.7868x; 1.7868x over previous
import functools

import jax
import jax.numpy as jnp
from jax import lax
from jax.experimental import pallas as pl
from jax.experimental.pallas import tpu as pltpu
from jax.experimental.pallas import tpu_sc as plsc

N_NODES_C = 10000
N_EDGES_C = 320000
CH = 128

_NC = 2
_NS = 16
_NW = _NC * _NS
_B = 128
_NCHUNK = 79
_EPT = _NCHUNK * _B
_NPAD = _NW * _EPT - N_EDGES_C
_NACC = N_NODES_C + _B


def _sc_body(x_hbm, row_hbm, col_hbm, zacc_hbm, zdeg_hbm,
             acc_out, deg_out,
             acc_sh, rbs, cbs, vs, degp, sems, semrs):
    c = lax.axis_index("c")
    s = lax.axis_index("s")
    wid = c * _NS + s
    base = pl.multiple_of(wid * _EPT, 8)

    pltpu.sync_copy(zdeg_hbm, degp)

    @pl.when(s == 0)
    def _():
        pltpu.sync_copy(zacc_hbm, acc_sh)

    plsc.subcore_barrier()

    ones16 = jnp.ones((16,), jnp.float32)

    def load_idx(j, b):
        off = pl.multiple_of(base + j * _B, 8)
        pltpu.async_copy(row_hbm.at[pl.ds(off, _B)], rbs[b], semrs[b])
        pltpu.async_copy(col_hbm.at[pl.ds(off, _B)], cbs[b], semrs[b])

    def wait_idx(j, b):
        off = pl.multiple_of(base + j * _B, 8)
        pltpu.make_async_copy(row_hbm.at[pl.ds(off, _B)], rbs[b], semrs[b]).wait()
        pltpu.make_async_copy(col_hbm.at[pl.ds(off, _B)], cbs[b], semrs[b]).wait()

    def gather(b, p):
        pltpu.async_copy(x_hbm.at[cbs[b]], vs[p], sems[p])

    def flush(b, p):
        rb = rbs[b]
        for i in range(_B // 16):
            plsc.addupdate_scatter(degp, [rb[pl.ds(i * 16, 16)]], ones16)
        pltpu.make_async_copy(x_hbm.at[cbs[b]], vs[p], sems[p]).wait()
        pltpu.sync_copy(vs[p], acc_sh.at[rb], add=True)

    for b in range(4):
        load_idx(b, b)
    wait_idx(0, 0)
    gather(0, 0)

    def step(k, carry):
        j = 4 * k
        for i in range(4):
            wait_idx(j + i + 1, (i + 1) % 4)
            gather((i + 1) % 4, (i + 1) % 2)
            flush(i % 4, i % 2)
            load_idx(j + i + 4, i % 4)
        return carry

    lax.fori_loop(0, (_NCHUNK - 3) // 4, step, 0)
    for i in range(3):
        if i < 2:
            wait_idx(76 + i + 1, i + 1)
            gather(i + 1, (i + 1) % 2)
        flush(i, i % 2)
    wait_idx(_NCHUNK, 3)

    d0 = pl.multiple_of(wid * _NACC, 8)
    pltpu.sync_copy(degp, deg_out.at[pl.ds(d0, _NACC)])

    plsc.subcore_barrier()

    @pl.when(s < 10)
    def _():
        for k in range(7):
            r0 = pl.multiple_of(s * 1000 + k * _B, 8)
            pltpu.sync_copy(acc_sh.at[pl.ds(r0, _B)], vs[0])
            pltpu.sync_copy(vs[0], acc_out.at[c, pl.ds(r0, _B)])
        r0 = pl.multiple_of(s * 1000 + 7 * _B, 8)
        pltpu.sync_copy(acc_sh.at[pl.ds(r0, 104)], vs[0].at[pl.ds(0, 104)])
        pltpu.sync_copy(vs[0].at[pl.ds(0, 104)], acc_out.at[c, pl.ds(r0, 104)])


@jax.jit
def _sc_scatter(x, row1, col1):
    zacc = jnp.zeros((_NACC, CH), jnp.float32)
    zdeg = jnp.zeros((_NACC,), jnp.float32)
    mesh = plsc.VectorSubcoreMesh(core_axis_name="c", subcore_axis_name="s")
    f = pl.kernel(
        _sc_body,
        out_type=[
            jax.ShapeDtypeStruct((_NC, N_NODES_C, CH), jnp.float32),
            jax.ShapeDtypeStruct((_NW * _NACC,), jnp.float32),
        ],
        mesh=mesh,
        compiler_params=pltpu.CompilerParams(needs_layout_passes=False),
        scratch_types=[
            pltpu.VMEM_SHARED((_NACC, CH), jnp.float32),
            [pltpu.VMEM((_B,), jnp.int32) for _ in range(4)],
            [pltpu.VMEM((_B,), jnp.int32) for _ in range(4)],
            [pltpu.VMEM((_B, CH), jnp.float32) for _ in range(2)],
            pltpu.VMEM((_NACC,), jnp.float32),
            [pltpu.SemaphoreType.DMA for _ in range(2)],
            [pltpu.SemaphoreType.DMA for _ in range(4)],
        ],
    )
    return f(x, row1, col1, zacc, zdeg)


def _tc_body(acc_ref, deg_ref, x_ref, w_ref, root_ref, bias_ref, o_ref):
    deg = jnp.maximum(jnp.sum(deg_ref[...], axis=1, keepdims=True), 1.0)
    agg = (acc_ref[0] + acc_ref[1]) / deg
    o_ref[...] = (
        jnp.dot(agg, w_ref[...], preferred_element_type=jnp.float32)
        + jnp.dot(x_ref[...], root_ref[...], preferred_element_type=jnp.float32)
        + bias_ref[...]
    )


@jax.jit
def _tc_combine(acc2, degt, x, weight, root, bias2):
    R = 1000
    grid = (N_NODES_C // R,)
    return pl.pallas_call(
        _tc_body,
        grid=grid,
        in_specs=[
            pl.BlockSpec((_NC, R, CH), lambda i: (0, i, 0)),
            pl.BlockSpec((R, _NW), lambda i: (i, 0)),
            pl.BlockSpec((R, CH), lambda i: (i, 0)),
            pl.BlockSpec((CH, CH), lambda i: (0, 0)),
            pl.BlockSpec((CH, CH), lambda i: (0, 0)),
            pl.BlockSpec((1, CH), lambda i: (0, 0)),
        ],
        out_specs=pl.BlockSpec((R, CH), lambda i: (i, 0)),
        out_shape=jax.ShapeDtypeStruct((N_NODES_C, CH), jnp.float32),
    )(acc2, degt, x, weight, root, bias2)


def kernel(x, edge_index, weight, root, bias):
    row = edge_index[0].astype(jnp.int32)
    col = edge_index[1].astype(jnp.int32)
    pad_ar = jnp.arange(_NPAD + _B, dtype=jnp.int32) % _B
    row1 = jnp.concatenate([row, N_NODES_C + pad_ar])
    col1 = jnp.concatenate([col, pad_ar])
    acc2, degf = _sc_scatter(x, row1, col1)
    degt = degf.reshape(_NW, _NACC).T
    return _tc_combine(acc2, degt, x, weight, root, bias[None, :])
